# Initial kernel scaffold; baseline (speedup 1.0000x reference)
import jax
import jax.numpy as jnp
from jax.experimental import pallas as pl


def kernel(x, edge_index, W1, b1, W2, b2):
    raise NotImplementedError("write your pallas kernel here")


# speedup vs baseline: 13.2949x; 13.2949x over previous
import functools
import jax
import jax.numpy as jnp
from jax import lax
from jax.experimental import pallas as pl
from jax.experimental.pallas import tpu as pltpu
from jax.experimental.pallas import tpu_sc as plsc

N = 10000
D = 128
DO = 2
DOP = 16
E = 320000
NPAD = 10240
NC = 2
NS = 16
NW = NC * NS
CH = 128
NCHUNK = 80
EPW = NCHUNK * CH
EPAD = NW * EPW
RPT = NPAD // NS
BLK = 1024
GRID = NPAD // BLK


def _mesh():
    return plsc.VectorSubcoreMesh(core_axis_name="c", subcore_axis_name="s")


@functools.partial(
    pl.kernel,
    out_type=jax.ShapeDtypeStruct((NPAD,), jnp.float32),
    mesh=_mesh(),
    scratch_types=[
        pltpu.VMEM((NCHUNK, CH), jnp.int32),
        pltpu.VMEM((CH,), jnp.float32),
        pltpu.VMEM((RPT,), jnp.float32),
        pltpu.VMEM_SHARED((NPAD,), jnp.float32),
    ],
)
def _deg_sc(dste, deg_out, didx, ones, zbuf, dacc):
    c = lax.axis_index("c")
    s = lax.axis_index("s")
    for k in range(CH // 16):
        ones[pl.ds(k * 16, 16)] = jnp.ones((16,), jnp.float32)
    for k in range(RPT // 16):
        zbuf[pl.ds(k * 16, 16)] = jnp.zeros((16,), jnp.float32)

    @pl.when(c == 0)
    def _():
        pltpu.sync_copy(zbuf, dacc.at[pl.ds(s * RPT, RPT)])
        plsc.subcore_barrier()
        for half in range(2):
            w = s * 2 + half
            pltpu.sync_copy(dste.at[w], didx)

            def body(j, _):
                pltpu.sync_copy(ones, dacc.at[didx.at[j]], add=True)
                return 0

            lax.fori_loop(0, NCHUNK, body, 0)
        plsc.subcore_barrier()
        pltpu.sync_copy(dacc.at[pl.ds(s * RPT, RPT)],
                        deg_out.at[pl.ds(s * RPT, RPT)])


@functools.partial(
    pl.kernel,
    out_type=jax.ShapeDtypeStruct((NC, NPAD, D), jnp.float32),
    mesh=_mesh(),
    scratch_types=[
        pltpu.VMEM((NCHUNK, CH), jnp.int32),
        pltpu.VMEM((NCHUNK, CH), jnp.int32),
        pltpu.VMEM((CH, D), jnp.float32),
        pltpu.VMEM_SHARED((NPAD, D), jnp.float32),
    ],
)
def _agg128_sc(y1, srce, dste, out, sidx, didx, gbuf, acc):
    c = lax.axis_index("c")
    s = lax.axis_index("s")
    w = s * NC + c

    def zrow(r, _):
        for k in range(D // 16):
            gbuf[r, pl.ds(k * 16, 16)] = jnp.zeros((16,), jnp.float32)
        return 0

    lax.fori_loop(0, CH, zrow, 0)
    for j in range(RPT // CH):
        pltpu.sync_copy(gbuf, acc.at[pl.ds(s * RPT + j * CH, CH)])
    pltpu.sync_copy(srce.at[w], sidx)
    pltpu.sync_copy(dste.at[w], didx)
    plsc.subcore_barrier()

    def body(j, _):
        pltpu.sync_copy(y1.at[sidx.at[j]], gbuf)
        pltpu.sync_copy(gbuf, acc.at[didx.at[j]], add=True)
        return 0

    lax.fori_loop(0, NCHUNK, body, 0)
    plsc.subcore_barrier()
    pltpu.sync_copy(acc.at[pl.ds(s * RPT, RPT)],
                    out.at[c, pl.ds(s * RPT, RPT)])


@functools.partial(
    pl.kernel,
    out_type=jax.ShapeDtypeStruct((NC, NPAD, DOP), jnp.float32),
    mesh=_mesh(),
    compiler_params=pltpu.CompilerParams(use_tc_tiling_on_sc=False),
    scratch_types=[
        pltpu.VMEM((NCHUNK, CH), jnp.int32),
        pltpu.VMEM((NCHUNK, CH), jnp.int32),
        pltpu.VMEM((CH, DOP), jnp.float32),
        pltpu.VMEM_SHARED((NPAD, DOP), jnp.float32),
    ],
)
def _agg16_sc(y2, srce, dste, out, sidx, didx, gbuf, acc):
    c = lax.axis_index("c")
    s = lax.axis_index("s")
    w = s * NC + c

    def zrow(r, _):
        gbuf[r, pl.ds(0, 16)] = jnp.zeros((16,), jnp.float32)
        return 0

    lax.fori_loop(0, CH, zrow, 0)
    for j in range(RPT // CH):
        pltpu.sync_copy(gbuf, acc.at[pl.ds(s * RPT + j * CH, CH)])
    pltpu.sync_copy(srce.at[w], sidx)
    pltpu.sync_copy(dste.at[w], didx)
    plsc.subcore_barrier()

    def body(j, _):
        pltpu.sync_copy(y2.at[sidx.at[j]], gbuf)
        pltpu.sync_copy(gbuf, acc.at[didx.at[j]], add=True)
        return 0

    lax.fori_loop(0, NCHUNK, body, 0)
    plsc.subcore_barrier()
    pltpu.sync_copy(acc.at[pl.ds(s * RPT, RPT)],
                    out.at[c, pl.ds(s * RPT, RPT)])


def _tc1_body(deg_ref, x_ref, w1_ref, y1_ref):
    dis = lax.rsqrt(deg_ref[...] + 1.0)
    xw = jnp.dot(x_ref[...], w1_ref[...], preferred_element_type=jnp.float32)
    y1_ref[...] = xw * dis


def _tc2_body(deg_ref, s1_ref, y1_ref, b1_ref, w2_ref, y2_ref):
    dis = lax.rsqrt(deg_ref[...] + 1.0)
    h = (s1_ref[0] + s1_ref[1] + y1_ref[...]) * dis + b1_ref[...]
    h = jnp.maximum(h, 0.0)
    y2_ref[...] = jnp.dot(h, w2_ref[...], preferred_element_type=jnp.float32) * dis


def _tc3_body(deg_ref, s2_ref, y2_ref, b2_ref, o_ref):
    dis = lax.rsqrt(deg_ref[...] + 1.0)
    o_ref[...] = (s2_ref[0] + s2_ref[1] + y2_ref[...]) * dis + b2_ref[...]


def kernel(x, edge_index, W1, b1, W2, b2):
    ei = edge_index.astype(jnp.int32)
    pad = jnp.full((EPAD - E,), N, dtype=jnp.int32)
    srce = jnp.concatenate([ei[0], pad]).reshape(NW, NCHUNK, CH)
    dste = jnp.concatenate([ei[1], pad]).reshape(NW, NCHUNK, CH)
    xp = jnp.pad(x, ((0, NPAD - N), (0, 0)))
    w2p = jnp.pad(W2, ((0, 0), (0, DOP - DO)))
    b1r = b1.reshape(1, D)
    b2r = jnp.pad(b2, (0, DOP - DO)).reshape(1, DOP)

    deg = _deg_sc(dste).reshape(NPAD, 1)

    y1 = pl.pallas_call(
        _tc1_body,
        grid=(GRID,),
        in_specs=[
            pl.BlockSpec((BLK, 1), lambda i: (i, 0)),
            pl.BlockSpec((BLK, D), lambda i: (i, 0)),
            pl.BlockSpec((D, D), lambda i: (0, 0)),
        ],
        out_specs=pl.BlockSpec((BLK, D), lambda i: (i, 0)),
        out_shape=jax.ShapeDtypeStruct((NPAD, D), jnp.float32),
    )(deg, xp, W1)

    s1 = _agg128_sc(y1, srce, dste)

    y2 = pl.pallas_call(
        _tc2_body,
        grid=(GRID,),
        in_specs=[
            pl.BlockSpec((BLK, 1), lambda i: (i, 0)),
            pl.BlockSpec((NC, BLK, D), lambda i: (0, i, 0)),
            pl.BlockSpec((BLK, D), lambda i: (i, 0)),
            pl.BlockSpec((1, D), lambda i: (0, 0)),
            pl.BlockSpec((D, DOP), lambda i: (0, 0)),
        ],
        out_specs=pl.BlockSpec((BLK, DOP), lambda i: (i, 0)),
        out_shape=jax.ShapeDtypeStruct((NPAD, DOP), jnp.float32),
    )(deg, s1, y1, b1r, w2p)

    s2 = _agg16_sc(y2, srce, dste)

    o = pl.pallas_call(
        _tc3_body,
        grid=(GRID,),
        in_specs=[
            pl.BlockSpec((BLK, 1), lambda i: (i, 0)),
            pl.BlockSpec((NC, BLK, DOP), lambda i: (0, i, 0)),
            pl.BlockSpec((BLK, DOP), lambda i: (i, 0)),
            pl.BlockSpec((1, DOP), lambda i: (0, 0)),
        ],
        out_specs=pl.BlockSpec((BLK, DOP), lambda i: (i, 0)),
        out_shape=jax.ShapeDtypeStruct((NPAD, DOP), jnp.float32),
    )(deg, s2, y2, b2r)

    return o[:N, :DO]


# speedup vs baseline: 13.6905x; 1.0298x over previous
import functools
import jax
import jax.numpy as jnp
from jax import lax
from jax.experimental import pallas as pl
from jax.experimental.pallas import tpu as pltpu
from jax.experimental.pallas import tpu_sc as plsc

N = 10000
D = 128
DO = 2
DOP = 16
E = 320000
NPAD = 10240
NC = 2
NS = 16
NW = NC * NS
CH = 80
NCHUNK = 128
EPW = NCHUNK * CH
EPAD = NW * EPW
RPT = NPAD // NS
BLK = 1024
GRID = NPAD // BLK

_SC_PARAMS = pltpu.CompilerParams(use_tc_tiling_on_sc=False)


def _mesh():
    return plsc.VectorSubcoreMesh(core_axis_name="c", subcore_axis_name="s")


@functools.partial(
    pl.kernel,
    out_type=jax.ShapeDtypeStruct((NPAD,), jnp.float32),
    mesh=_mesh(),
    compiler_params=_SC_PARAMS,
    scratch_types=[
        pltpu.VMEM((NCHUNK, CH), jnp.int32),
        pltpu.VMEM((CH,), jnp.float32),
        pltpu.VMEM((RPT,), jnp.float32),
        pltpu.VMEM_SHARED((NPAD,), jnp.float32),
        pltpu.SemaphoreType.DMA,
    ],
)
def _deg_sc(dste, deg_out, didx, ones, zbuf, dacc, dsem):
    c = lax.axis_index("c")
    s = lax.axis_index("s")
    for k in range(CH // 16):
        ones[pl.ds(k * 16, 16)] = jnp.ones((16,), jnp.float32)
    for k in range(RPT // 16):
        zbuf[pl.ds(k * 16, 16)] = jnp.zeros((16,), jnp.float32)

    @pl.when(c == 0)
    def _():
        pltpu.sync_copy(zbuf, dacc.at[pl.ds(s * RPT, RPT)])
        plsc.subcore_barrier()
        for half in range(2):
            w = s * 2 + half
            pltpu.sync_copy(dste.at[w], didx)

            def body(p, _):
                for b in range(8):
                    pltpu.async_copy(ones, dacc.at[didx.at[p * 8 + b]],
                                     dsem, add=True)
                for b in range(8):
                    pltpu.make_async_copy(ones,
                                          dacc.at[didx.at[p * 8 + b]],
                                          dsem).wait()
                return 0

            lax.fori_loop(0, NCHUNK // 8, body, 0)
        plsc.subcore_barrier()
        pltpu.sync_copy(dacc.at[pl.ds(s * RPT, RPT)],
                        deg_out.at[pl.ds(s * RPT, RPT)])


def _agg_body(y, srce, dste, out, sidx, didx, gbuf, acc,
              g0, g1, s0, s1, width):
    c = lax.axis_index("c")
    s = lax.axis_index("s")
    w = s * NC + c
    npairs = NCHUNK // 2

    def zrow(r, _):
        for k in range(width // 16):
            gbuf[0, r, pl.ds(k * 16, 16)] = jnp.zeros((16,), jnp.float32)
        return 0

    lax.fori_loop(0, CH, zrow, 0)
    for j in range(RPT // CH):
        pltpu.sync_copy(gbuf.at[0], acc.at[pl.ds(s * RPT + j * CH, CH)])
    pltpu.sync_copy(srce.at[w], sidx)
    pltpu.sync_copy(dste.at[w], didx)
    plsc.subcore_barrier()

    gsem = (g0, g1)
    ssem = (s0, s1)

    def wait_g(j, b):
        pltpu.make_async_copy(y.at[sidx.at[j]], gbuf.at[b], gsem[b]).wait()

    def fire_g(j, b):
        pltpu.async_copy(y.at[sidx.at[j]], gbuf.at[b], gsem[b])

    def wait_s(j, b):
        pltpu.make_async_copy(gbuf.at[b], acc.at[didx.at[j]], ssem[b]).wait()

    def fire_s(j, b):
        pltpu.async_copy(gbuf.at[b], acc.at[didx.at[j]], ssem[b], add=True)

    fire_g(0, 0)

    def grp(p, _):
        j0 = p * 2
        j1 = j0 + 1
        wait_g(j0, 0)
        fire_s(j0, 0)

        @pl.when(p > 0)
        def _():
            wait_s(j0 - 1, 1)

        fire_g(j1, 1)
        wait_g(j1, 1)
        fire_s(j1, 1)
        wait_s(j0, 0)

        @pl.when(p < npairs - 1)
        def _():
            fire_g(j0 + 2, 0)

        return 0

    lax.fori_loop(0, npairs, grp, 0)
    wait_s(NCHUNK - 1, 1)
    plsc.subcore_barrier()
    pltpu.sync_copy(acc.at[pl.ds(s * RPT, RPT)],
                    out.at[c, pl.ds(s * RPT, RPT)])


@functools.partial(
    pl.kernel,
    out_type=jax.ShapeDtypeStruct((NC, NPAD, D), jnp.float32),
    mesh=_mesh(),
    compiler_params=_SC_PARAMS,
    scratch_types=[
        pltpu.VMEM((NCHUNK, CH), jnp.int32),
        pltpu.VMEM((NCHUNK, CH), jnp.int32),
        pltpu.VMEM((2, CH, D), jnp.float32),
        pltpu.VMEM_SHARED((NPAD, D), jnp.float32),
        pltpu.SemaphoreType.DMA,
        pltpu.SemaphoreType.DMA,
        pltpu.SemaphoreType.DMA,
        pltpu.SemaphoreType.DMA,
    ],
)
def _agg128_sc(y1, srce, dste, out, sidx, didx, gbuf, acc, g0, g1, s0, s1):
    _agg_body(y1, srce, dste, out, sidx, didx, gbuf, acc, g0, g1, s0, s1, D)


@functools.partial(
    pl.kernel,
    out_type=jax.ShapeDtypeStruct((NC, NPAD, DOP), jnp.float32),
    mesh=_mesh(),
    compiler_params=_SC_PARAMS,
    scratch_types=[
        pltpu.VMEM((NCHUNK, CH), jnp.int32),
        pltpu.VMEM((NCHUNK, CH), jnp.int32),
        pltpu.VMEM((2, CH, DOP), jnp.float32),
        pltpu.VMEM_SHARED((NPAD, DOP), jnp.float32),
        pltpu.SemaphoreType.DMA,
        pltpu.SemaphoreType.DMA,
        pltpu.SemaphoreType.DMA,
        pltpu.SemaphoreType.DMA,
    ],
)
def _agg16_sc(y2, srce, dste, out, sidx, didx, gbuf, acc, g0, g1, s0, s1):
    _agg_body(y2, srce, dste, out, sidx, didx, gbuf, acc, g0, g1, s0, s1, DOP)


def _tc1_body(deg_ref, x_ref, w1_ref, y1_ref):
    dis = lax.rsqrt(deg_ref[...] + 1.0)
    xw = jnp.dot(x_ref[...], w1_ref[...], preferred_element_type=jnp.float32)
    y1_ref[...] = xw * dis


def _tc2_body(deg_ref, s1_ref, y1_ref, b1_ref, w2_ref, y2_ref):
    dis = lax.rsqrt(deg_ref[...] + 1.0)
    h = (s1_ref[0] + s1_ref[1] + y1_ref[...]) * dis + b1_ref[...]
    h = jnp.maximum(h, 0.0)
    y2_ref[...] = jnp.dot(h, w2_ref[...], preferred_element_type=jnp.float32) * dis


def _tc3_body(deg_ref, s2_ref, y2_ref, b2_ref, o_ref):
    dis = lax.rsqrt(deg_ref[...] + 1.0)
    o_ref[...] = (s2_ref[0] + s2_ref[1] + y2_ref[...]) * dis + b2_ref[...]


def kernel(x, edge_index, W1, b1, W2, b2):
    ei = edge_index.astype(jnp.int32)
    pad = jnp.full((EPAD - E,), N, dtype=jnp.int32)
    srce = jnp.concatenate([ei[0], pad]).reshape(NW, NCHUNK, CH)
    dste = jnp.concatenate([ei[1], pad]).reshape(NW, NCHUNK, CH)
    xp = jnp.pad(x, ((0, NPAD - N), (0, 0)))
    w2p = jnp.pad(W2, ((0, 0), (0, DOP - DO)))
    b1r = b1.reshape(1, D)
    b2r = jnp.pad(b2, (0, DOP - DO)).reshape(1, DOP)

    deg = _deg_sc(dste).reshape(NPAD, 1)

    y1 = pl.pallas_call(
        _tc1_body,
        grid=(GRID,),
        in_specs=[
            pl.BlockSpec((BLK, 1), lambda i: (i, 0)),
            pl.BlockSpec((BLK, D), lambda i: (i, 0)),
            pl.BlockSpec((D, D), lambda i: (0, 0)),
        ],
        out_specs=pl.BlockSpec((BLK, D), lambda i: (i, 0)),
        out_shape=jax.ShapeDtypeStruct((NPAD, D), jnp.float32),
    )(deg, xp, W1)

    s1 = _agg128_sc(y1, srce, dste)

    y2 = pl.pallas_call(
        _tc2_body,
        grid=(GRID,),
        in_specs=[
            pl.BlockSpec((BLK, 1), lambda i: (i, 0)),
            pl.BlockSpec((NC, BLK, D), lambda i: (0, i, 0)),
            pl.BlockSpec((BLK, D), lambda i: (i, 0)),
            pl.BlockSpec((1, D), lambda i: (0, 0)),
            pl.BlockSpec((D, DOP), lambda i: (0, 0)),
        ],
        out_specs=pl.BlockSpec((BLK, DOP), lambda i: (i, 0)),
        out_shape=jax.ShapeDtypeStruct((NPAD, DOP), jnp.float32),
    )(deg, s1, y1, b1r, w2p)

    s2 = _agg16_sc(y2, srce, dste)

    o = pl.pallas_call(
        _tc3_body,
        grid=(GRID,),
        in_specs=[
            pl.BlockSpec((BLK, 1), lambda i: (i, 0)),
            pl.BlockSpec((NC, BLK, DOP), lambda i: (0, i, 0)),
            pl.BlockSpec((BLK, DOP), lambda i: (i, 0)),
            pl.BlockSpec((1, DOP), lambda i: (0, 0)),
        ],
        out_specs=pl.BlockSpec((BLK, DOP), lambda i: (i, 0)),
        out_shape=jax.ShapeDtypeStruct((NPAD, DOP), jnp.float32),
    )(deg, s2, y2, b2r)

    return o[:N, :DO]
